# baseline (speedup 1.0000x reference)
import jax
import jax.numpy as jnp
from jax.experimental import pallas as pl
from jax.experimental.pallas import tpu as pltpu

_VMEM_LIMIT = 64 << 20


def _silu(x):
    return x * (1.0 / (1.0 + jnp.exp(-x)))


def _make_fused_kernel(H, W, c_):
    HW = H * W

    def _body(x_ref, w12_ref, b12_ref, w1_ref, b1_ref, key_ref, kb_ref,
              val_ref, vb_ref, a1_ref, a1b_ref, a2_ref, a2b_ref,
              w3_ref, b3_ref, o_ref):
        xb = x_ref[0].astype(jnp.bfloat16)

        ab = jnp.dot(w12_ref[...], xb,
                     preferred_element_type=jnp.float32) + b12_ref[...]
        ab = _silu(ab)
        ab_b = ab.astype(jnp.bfloat16)
        x_in = ab[:c_]

        z = jnp.dot(w1_ref[...], ab_b[:c_],
                    preferred_element_type=jnp.float32) + b1_ref[...]
        z = _silu(z)
        zb = z.astype(jnp.bfloat16)

        zp = jnp.concatenate(
            [jnp.zeros((c_, W + 1), jnp.bfloat16), zb,
             jnp.zeros((c_, W + 1), jnp.bfloat16)], axis=1)
        col = jax.lax.broadcasted_iota(jnp.int32, (1, HW), 1) % W
        m_dx = {
            -1: (col != 0),
            0: None,
            1: (col != W - 1),
        }
        taps = []
        for dy in (-1, 0, 1):
            for dx in (-1, 0, 1):
                s = dy * W + dx
                t = jax.lax.slice(zp, (0, W + 1 + s), (c_, W + 1 + s + HW))
                m = m_dx[dx]
                if m is not None:
                    t = jnp.where(m, t, jnp.bfloat16(0))
                taps.append(t)
        im2col = jnp.concatenate(taps, axis=0)
        k1 = jnp.dot(key_ref[...], im2col,
                     preferred_element_type=jnp.float32) + kb_ref[...]
        k1 = jnp.maximum(k1, 0.0)
        k1b = k1.astype(jnp.bfloat16)

        v = jnp.dot(val_ref[...], zb,
                    preferred_element_type=jnp.float32) + vb_ref[...]

        hid = jnp.dot(a1_ref[...], jnp.concatenate([k1b, zb], axis=0),
                      preferred_element_type=jnp.float32) + a1b_ref[...]
        hid = jnp.maximum(hid, 0.0)
        att = jnp.dot(a2_ref[...], hid.astype(jnp.bfloat16),
                      preferred_element_type=jnp.float32) + a2b_ref[...]

        mx = jnp.max(att, axis=1, keepdims=True)
        e = jnp.exp(att - mx)
        s = jnp.sum(e, axis=1, keepdims=True)
        inv = pl.reciprocal(s, approx=True)
        m_out = x_in + k1 + (e * inv) * v

        cat3 = jnp.concatenate([m_out.astype(jnp.bfloat16), ab_b[c_:]], axis=0)
        out = jnp.dot(w3_ref[...], cat3,
                      preferred_element_type=jnp.float32) + b3_ref[...]
        o_ref[0] = _silu(out).astype(o_ref.dtype)

    return _body


def kernel(x, cv12_w, cv12_b, cv3_wm, cv3_wab, cv3_b, m0_cv1_w, m0_cv1_b,
           m0_key_w, m0_key_b, m0_val_w, m0_val_b, m0_att1_wk, m0_att1_wz,
           m0_att1_b, m0_att2_w, m0_att2_b):
    N, C1, H, W = x.shape
    HW = H * W
    c_ = m0_cv1_b.shape[1]
    C2 = cv3_b.shape[1]

    bf = jnp.bfloat16
    w12 = cv12_w.T.astype(bf)
    w1 = m0_cv1_w.T.astype(bf)
    keyw = m0_key_w.T.astype(bf)
    valw = m0_val_w.T.astype(bf)
    a1 = jnp.concatenate([m0_att1_wk, m0_att1_wz], axis=0).T.astype(bf)
    a2 = m0_att2_w.T.astype(bf)
    w3 = jnp.concatenate([cv3_wm, cv3_wab[c_:]], axis=0).T.astype(bf)

    b12 = cv12_b.T
    b1 = m0_cv1_b.T
    kb = m0_key_b.T
    vb = m0_val_b.T
    a1b = m0_att1_b.T
    a2b = m0_att2_b.T
    b3 = cv3_b.T

    x3 = x.reshape(N, C1, HW)

    def const(a):
        return pl.BlockSpec(a.shape, lambda n: (0, 0))

    out = pl.pallas_call(
        _make_fused_kernel(H, W, c_),
        out_shape=jax.ShapeDtypeStruct((N, C2, HW), x.dtype),
        grid_spec=pltpu.PrefetchScalarGridSpec(
            num_scalar_prefetch=0,
            grid=(N,),
            in_specs=[
                pl.BlockSpec((1, C1, HW), lambda n: (n, 0, 0)),
                const(w12), const(b12), const(w1), const(b1),
                const(keyw), const(kb), const(valw), const(vb),
                const(a1), const(a1b), const(a2), const(a2b),
                const(w3), const(b3),
            ],
            out_specs=pl.BlockSpec((1, C2, HW), lambda n: (n, 0, 0)),
        ),
        compiler_params=pltpu.CompilerParams(
            dimension_semantics=("parallel",), vmem_limit_bytes=_VMEM_LIMIT),
    )(x3, w12, b12, w1, b1, keyw, kb, valw, vb, a1, a1b, a2, a2b, w3, b3)
    return out.reshape(N, C2, H, W)


# speedup vs baseline: 1.0195x; 1.0195x over previous
import jax
import jax.numpy as jnp
from jax.experimental import pallas as pl
from jax.experimental.pallas import tpu as pltpu

_VMEM_LIMIT = 64 << 20


def _silu(x):
    return x * pl.reciprocal(1.0 + jnp.exp(-x), approx=True)


def _make_fused_kernel(H, W, c_, imgs):
    HW = H * W

    def _one_image(x_img, w12_ref, b12_ref, w1_ref, b1_ref, key_ref, kb_ref,
                   val_ref, vb_ref, a1_ref, a1b_ref, a2_ref, a2b_ref,
                   w3_ref, b3_ref):
        xb = x_img.astype(jnp.bfloat16)

        ab = jnp.dot(w12_ref[...], xb,
                     preferred_element_type=jnp.float32) + b12_ref[...]
        ab = _silu(ab)
        ab_b = ab.astype(jnp.bfloat16)
        x_in = ab[:c_]

        z = jnp.dot(w1_ref[...], ab_b[:c_],
                    preferred_element_type=jnp.float32) + b1_ref[...]
        z = _silu(z)
        zb = z.astype(jnp.bfloat16)

        zp = jnp.concatenate(
            [jnp.zeros((c_, W + 1), jnp.bfloat16), zb,
             jnp.zeros((c_, W + 1), jnp.bfloat16)], axis=1)
        col = jax.lax.broadcasted_iota(jnp.int32, (1, HW), 1) % W
        m_dx = {
            -1: (col != 0),
            0: None,
            1: (col != W - 1),
        }
        taps = []
        for dy in (-1, 0, 1):
            for dx in (-1, 0, 1):
                s = dy * W + dx
                t = jax.lax.slice(zp, (0, W + 1 + s), (c_, W + 1 + s + HW))
                m = m_dx[dx]
                if m is not None:
                    t = jnp.where(m, t, jnp.bfloat16(0))
                taps.append(t)
        im2col = jnp.concatenate(taps, axis=0)
        k1 = jnp.dot(key_ref[...], im2col,
                     preferred_element_type=jnp.float32) + kb_ref[...]
        k1 = jnp.maximum(k1, 0.0)
        k1b = k1.astype(jnp.bfloat16)

        v = jnp.dot(val_ref[...], zb,
                    preferred_element_type=jnp.float32) + vb_ref[...]

        hid = jnp.dot(a1_ref[...], jnp.concatenate([k1b, zb], axis=0),
                      preferred_element_type=jnp.float32) + a1b_ref[...]
        hid = jnp.maximum(hid, 0.0)
        att = jnp.dot(a2_ref[...], hid.astype(jnp.bfloat16),
                      preferred_element_type=jnp.float32) + a2b_ref[...]

        mx = jnp.max(att, axis=1, keepdims=True)
        e = jnp.exp(att - mx)
        s = jnp.sum(e, axis=1, keepdims=True)
        inv = pl.reciprocal(s, approx=True)
        m_out = x_in + k1 + (e * inv) * v

        cat3 = jnp.concatenate([m_out.astype(jnp.bfloat16), ab_b[c_:]], axis=0)
        out = jnp.dot(w3_ref[...], cat3,
                      preferred_element_type=jnp.float32) + b3_ref[...]
        return _silu(out)

    def _body(x_ref, w12_ref, b12_ref, w1_ref, b1_ref, key_ref, kb_ref,
              val_ref, vb_ref, a1_ref, a1b_ref, a2_ref, a2b_ref,
              w3_ref, b3_ref, o_ref):
        for i in range(imgs):
            out = _one_image(x_ref[i], w12_ref, b12_ref, w1_ref, b1_ref,
                             key_ref, kb_ref, val_ref, vb_ref, a1_ref,
                             a1b_ref, a2_ref, a2b_ref, w3_ref, b3_ref)
            o_ref[i] = out.astype(o_ref.dtype)

    return _body


def kernel(x, cv12_w, cv12_b, cv3_wm, cv3_wab, cv3_b, m0_cv1_w, m0_cv1_b,
           m0_key_w, m0_key_b, m0_val_w, m0_val_b, m0_att1_wk, m0_att1_wz,
           m0_att1_b, m0_att2_w, m0_att2_b):
    N, C1, H, W = x.shape
    HW = H * W
    c_ = m0_cv1_b.shape[1]
    C2 = cv3_b.shape[1]
    IMGS = 2 if N % 2 == 0 else 1

    bf = jnp.bfloat16
    w12 = cv12_w.T.astype(bf)
    w1 = m0_cv1_w.T.astype(bf)
    keyw = m0_key_w.T.astype(bf)
    valw = m0_val_w.T.astype(bf)
    a1 = jnp.concatenate([m0_att1_wk, m0_att1_wz], axis=0).T.astype(bf)
    a2 = m0_att2_w.T.astype(bf)
    w3 = jnp.concatenate([cv3_wm, cv3_wab[c_:]], axis=0).T.astype(bf)

    b12 = cv12_b.T
    b1 = m0_cv1_b.T
    kb = m0_key_b.T
    vb = m0_val_b.T
    a1b = m0_att1_b.T
    a2b = m0_att2_b.T
    b3 = cv3_b.T

    x3 = x.reshape(N, C1, HW)

    def const(a):
        return pl.BlockSpec(a.shape, lambda n: (0, 0))

    out = pl.pallas_call(
        _make_fused_kernel(H, W, c_, IMGS),
        out_shape=jax.ShapeDtypeStruct((N, C2, HW), x.dtype),
        grid_spec=pltpu.PrefetchScalarGridSpec(
            num_scalar_prefetch=0,
            grid=(N // IMGS,),
            in_specs=[
                pl.BlockSpec((IMGS, C1, HW), lambda n: (n, 0, 0)),
                const(w12), const(b12), const(w1), const(b1),
                const(keyw), const(kb), const(valw), const(vb),
                const(a1), const(a1b), const(a2), const(a2b),
                const(w3), const(b3),
            ],
            out_specs=pl.BlockSpec((IMGS, C2, HW), lambda n: (n, 0, 0)),
        ),
        compiler_params=pltpu.CompilerParams(
            dimension_semantics=("parallel",), vmem_limit_bytes=_VMEM_LIMIT),
    )(x3, w12, b12, w1, b1, keyw, kb, valw, vb, a1, a1b, a2, a2b, w3, b3)
    return out.reshape(N, C2, H, W)
